# baseline (speedup 1.0000x reference)
import functools

import jax
import jax.numpy as jnp
from jax import lax
from jax.experimental import pallas as pl
from jax.experimental.pallas import tpu as pltpu
from jax.experimental.pallas import tpu_sc as plsc

N = 10000
E = 320000
D = 128
R = 8
B = 4

NC = 2
NS = 16
NW = NC * NS
E_PER_W = E // NW
CHUNK = 80
N_CHUNKS = E_PER_W // CHUNK
NH = N // 2
ACC_ROWS = NH + 8
ROWS_PER_SUB = 312
ROWS_TAIL = ACC_ROWS - NS * ROWS_PER_SUB
CNT_PER_SUB = (R * N) // NS

_mesh = plsc.VectorSubcoreMesh(core_axis_name="c", subcore_axis_name="s")


@functools.partial(
    pl.kernel, mesh=_mesh,
    out_type=jax.ShapeDtypeStruct((NC * R * N,), jnp.float32),
    scratch_types=[
        pltpu.VMEM((CHUNK,), jnp.int32),
        pltpu.VMEM((CHUNK,), jnp.float32),
        pltpu.VMEM((CNT_PER_SUB,), jnp.float32),
        pltpu.VMEM_SHARED((R * N,), jnp.float32),
    ],
)
def _count_kernel(seg_hbm, ones_hbm, zeros_hbm, out_hbm, idx_v, ones_v, b_v, cnt_sh):
    c = lax.axis_index("c")
    s = lax.axis_index("s")
    wid = s * NC + c
    pltpu.sync_copy(zeros_hbm.at[pl.ds(s * CNT_PER_SUB, CNT_PER_SUB)], b_v)
    pltpu.sync_copy(b_v, cnt_sh.at[pl.ds(s * CNT_PER_SUB, CNT_PER_SUB)])
    pltpu.sync_copy(ones_hbm, ones_v)
    plsc.subcore_barrier()
    for i in range(N_CHUNKS):
        base = wid * E_PER_W + i * CHUNK
        pltpu.sync_copy(seg_hbm.at[pl.ds(base, CHUNK)], idx_v)
        pltpu.sync_copy(ones_v, cnt_sh.at[idx_v], add=True)
    plsc.subcore_barrier()
    pltpu.sync_copy(cnt_sh.at[pl.ds(s * CNT_PER_SUB, CNT_PER_SUB)], b_v)
    pltpu.sync_copy(b_v, out_hbm.at[pl.ds(c * R * N + s * CNT_PER_SUB, CNT_PER_SUB)])


@functools.partial(
    pl.kernel, mesh=_mesh,
    out_type=jax.ShapeDtypeStruct((2, NC, ACC_ROWS, D), jnp.float32),
    scratch_types=[
        pltpu.VMEM((CHUNK,), jnp.int32),
        pltpu.VMEM((CHUNK,), jnp.int32),
        pltpu.VMEM((CHUNK,), jnp.int32),
        pltpu.VMEM((CHUNK,), jnp.int32),
        pltpu.VMEM((CHUNK,), jnp.int32),
        pltpu.VMEM((CHUNK, D), jnp.float32),
        pltpu.VMEM((CHUNK, D), jnp.float32),
        pltpu.VMEM((104, D), jnp.float32),
        pltpu.VMEM_SHARED((ACC_ROWS, D), jnp.float32),
        pltpu.SemaphoreType.DMA,
    ],
)
def _edge_kernel(table_hbm, gidx_hbm, dst_hbm, seg_hbm, winv_hbm, zeros_hbm, out_hbm,
                 idx_v, dst_v, seg_v, dstc_v, segc_v, rows_v, wrows_v, b_v,
                 acc_sh, sem):
    c = lax.axis_index("c")
    s = lax.axis_index("s")
    wid = s * NC + c
    pltpu.sync_copy(zeros_hbm.at[pl.ds(0, 104)], b_v)
    for h in range(2):
        lo = h * NH
        for k in range(3):
            pltpu.sync_copy(b_v, acc_sh.at[pl.ds(s * ROWS_PER_SUB + k * 104, 104)])

        @pl.when(s == 0)
        def _zero_tail():
            pltpu.sync_copy(b_v.at[pl.ds(0, ROWS_TAIL)],
                            acc_sh.at[pl.ds(NS * ROWS_PER_SUB, ROWS_TAIL)])

        plsc.subcore_barrier()

        def _chunk(i, carry):
            base = pl.multiple_of(wid * E_PER_W + i * CHUNK, 8)
            pltpu.sync_copy(gidx_hbm.at[pl.ds(base, CHUNK)], idx_v)
            pltpu.sync_copy(dst_hbm.at[pl.ds(base, CHUNK)], dst_v)
            pltpu.sync_copy(seg_hbm.at[pl.ds(base, CHUNK)], seg_v)
            dst = dst_v[...]
            valid = jnp.logical_and(dst >= lo, dst < lo + NH)
            dstc_v[...] = jnp.where(valid, dst - lo, NH)
            segc_v[...] = jnp.where(valid, seg_v[...], R * N)
            pltpu.async_copy(table_hbm.at[idx_v], rows_v, sem).wait()
            pltpu.async_copy(winv_hbm.at[segc_v], wrows_v, sem).wait()
            rows_v[...] = rows_v[...] * wrows_v[...]
            pltpu.sync_copy(rows_v, acc_sh.at[dstc_v], add=True)
            return carry

        lax.fori_loop(0, N_CHUNKS, _chunk, 0)
        plsc.subcore_barrier()
        for k in range(3):
            pltpu.sync_copy(acc_sh.at[pl.ds(s * ROWS_PER_SUB + k * 104, 104)], b_v)
            pltpu.sync_copy(b_v, out_hbm.at[h, c, pl.ds(s * ROWS_PER_SUB + k * 104, 104)])

        @pl.when(s == 0)
        def _out_tail():
            pltpu.sync_copy(acc_sh.at[pl.ds(NS * ROWS_PER_SUB, ROWS_TAIL)],
                            b_v.at[pl.ds(0, ROWS_TAIL)])
            pltpu.sync_copy(b_v.at[pl.ds(0, ROWS_TAIL)],
                            out_hbm.at[h, c, pl.ds(NS * ROWS_PER_SUB, ROWS_TAIL)])

        plsc.subcore_barrier()
        pltpu.sync_copy(zeros_hbm.at[pl.ds(0, 104)], b_v)


_MM_BLK = 400


def _mm_body(x_ref, bases_ref, comp_ref, root_ref, out_ref):
    xb = x_ref[...]
    comp = comp_ref[...]
    bases = bases_ref[...]
    w = comp[0, 0, 0] * bases[0]
    for b in range(1, B):
        w = w + comp[0, 0, b] * bases[b]
    is_root = jnp.where(pl.program_id(0) == R, 1.0, 0.0)
    w = w + is_root * root_ref[...]
    out_ref[...] = jnp.dot(xb, w, preferred_element_type=jnp.float32)


def _mm(x, bases, comp3, root):
    nblk = N // _MM_BLK
    return pl.pallas_call(
        _mm_body,
        grid=(R + 1, nblk),
        in_specs=[
            pl.BlockSpec((_MM_BLK, D), lambda r, i: (i, 0)),
            pl.BlockSpec((B, D, D), lambda r, i: (0, 0, 0)),
            pl.BlockSpec((1, 1, B), lambda r, i: (r, 0, 0)),
            pl.BlockSpec((D, D), lambda r, i: (0, 0)),
        ],
        out_specs=pl.BlockSpec((_MM_BLK, D), lambda r, i: (r * (N // _MM_BLK) + i, 0)),
        out_shape=jax.ShapeDtypeStruct(((R + 1) * N, D), jnp.float32),
    )(x, bases, comp3, root)


def _combine_body(do_relu, a0_ref, a1_ref, rp_ref, bias_ref, out_ref):
    t = a0_ref[...] + a1_ref[...] + rp_ref[...] + bias_ref[...]
    if do_relu:
        t = jnp.maximum(t, 0.0)
    out_ref[...] = t


def _combine(a0, a1, rp, bias2d, do_relu):
    grid = (N // _MM_BLK,)
    return pl.pallas_call(
        functools.partial(_combine_body, do_relu),
        grid=grid,
        in_specs=[
            pl.BlockSpec((_MM_BLK, D), lambda i: (i, 0)),
            pl.BlockSpec((_MM_BLK, D), lambda i: (i, 0)),
            pl.BlockSpec((_MM_BLK, D), lambda i: (i, 0)),
            pl.BlockSpec((1, D), lambda i: (0, 0)),
        ],
        out_specs=pl.BlockSpec((_MM_BLK, D), lambda i: (i, 0)),
        out_shape=jax.ShapeDtypeStruct((N, D), jnp.float32),
    )(a0, a1, rp, bias2d)


def _layer(h, bases, comp3, root, bias, gidx, dst, seg, winv, zeros_nd, do_relu):
    table = _mm(h, bases, comp3, root)
    rootp = table[R * N:(R + 1) * N]
    parts = _edge_kernel(table, gidx, dst, seg, winv, zeros_nd)
    a0 = jnp.concatenate([parts[0, 0, :NH], parts[1, 0, :NH]], axis=0)
    a1 = jnp.concatenate([parts[0, 1, :NH], parts[1, 1, :NH]], axis=0)
    return _combine(a0, a1, rootp, bias.reshape(1, D), do_relu)


def kernel(x, edge_index, edge_type, bases1, comp1, root1, bias1,
           bases2, comp2, root2, bias2, bases3, comp3, root3, bias3):
    src = edge_index[0].astype(jnp.int32)
    dst = edge_index[1].astype(jnp.int32)
    etype = edge_type.astype(jnp.int32)

    gidx = etype * N + src
    seg = etype * N + dst

    ones_c = jnp.ones((CHUNK,), jnp.float32)
    zeros_cnt = jnp.zeros((R * N,), jnp.float32)
    zeros_nd = jnp.zeros((104, D), jnp.float32)

    cnt_flat = _count_kernel(seg, ones_c, zeros_cnt)
    count = cnt_flat[:R * N] + cnt_flat[R * N:]
    inv = 1.0 / jnp.maximum(count, 1.0)
    winv = jnp.concatenate(
        [jnp.broadcast_to(inv[:, None], (R * N, D)),
         jnp.zeros((8, D), jnp.float32)], axis=0)

    pad = jnp.zeros((1, 1, B), jnp.float32)
    c1 = jnp.concatenate([comp1.reshape(R, 1, B), pad], axis=0)
    c2 = jnp.concatenate([comp2.reshape(R, 1, B), pad], axis=0)
    c3 = jnp.concatenate([comp3.reshape(R, 1, B), pad], axis=0)

    h = _layer(x, bases1, c1, root1, bias1, gidx, dst, seg, winv, zeros_nd, True)
    h = _layer(h, bases2, c2, root2, bias2, gidx, dst, seg, winv, zeros_nd, True)
    h = _layer(h, bases3, c3, root3, bias3, gidx, dst, seg, winv, zeros_nd, False)
    return h
